# Initial kernel scaffold; baseline (speedup 1.0000x reference)
import jax
import jax.numpy as jnp
from jax import lax
from jax.experimental import pallas as pl

X_BLK = 16
N = 256


def _body(x_ref, o_ref):
    x = x_ref[...]
    z1 = lax.dynamic_slice_in_dim(x, 1, 1, axis=2)
    zi = lax.broadcasted_iota(jnp.int32, x.shape, 2)
    o_ref[...] = jnp.where(zi == 0, z1, x)


def kernel(b):
    bs = b.reshape(N, N, N)
    out = pl.pallas_call(
        _body,
        grid=(N // X_BLK,),
        in_specs=[pl.BlockSpec((X_BLK, N, N), lambda i: (i, 0, 0))],
        out_specs=pl.BlockSpec((X_BLK, N, N), lambda i: (i, 0, 0)),
        out_shape=jax.ShapeDtypeStruct((N, N, N), jnp.float32),
    )(bs)
    return out.reshape(1, N, N, N, 1)


# speedup vs baseline: 6.8985x; 6.8985x over previous
import jax
import jax.numpy as jnp
from jax import lax
from jax.experimental import pallas as pl

X_BLK = 16
N = 256


def _body(x_ref, o_ref):
    x = x_ref[...]
    z1 = x[:, :, 1:2]
    zi = lax.broadcasted_iota(jnp.int32, x.shape, 2)
    o_ref[...] = jnp.where(zi == 0, z1, x)


def kernel(b):
    bs = b.reshape(N, N, N)
    out = pl.pallas_call(
        _body,
        grid=(N // X_BLK,),
        in_specs=[pl.BlockSpec((X_BLK, N, N), lambda i: (i, 0, 0))],
        out_specs=pl.BlockSpec((X_BLK, N, N), lambda i: (i, 0, 0)),
        out_shape=jax.ShapeDtypeStruct((N, N, N), jnp.float32),
    )(bs)
    return out.reshape(1, N, N, N, 1)


# speedup vs baseline: 6.9477x; 1.0071x over previous
import jax
import jax.numpy as jnp
from jax import lax
from jax.experimental import pallas as pl

X_BLK = 32
N = 256


def _body(x_ref, o_ref):
    x = x_ref[...]
    z1 = x[:, :, 1:2]
    zi = lax.broadcasted_iota(jnp.int32, x.shape, 2)
    o_ref[...] = jnp.where(zi == 0, z1, x)


def kernel(b):
    bs = b.reshape(N, N, N)
    out = pl.pallas_call(
        _body,
        grid=(N // X_BLK,),
        in_specs=[pl.BlockSpec((X_BLK, N, N), lambda i: (i, 0, 0))],
        out_specs=pl.BlockSpec((X_BLK, N, N), lambda i: (i, 0, 0)),
        out_shape=jax.ShapeDtypeStruct((N, N, N), jnp.float32),
    )(bs)
    return out.reshape(1, N, N, N, 1)


# speedup vs baseline: 6.9912x; 1.0063x over previous
import jax
import jax.numpy as jnp
from jax import lax
from jax.experimental import pallas as pl
from jax.experimental.pallas import tpu as pltpu

N = 256
R = N * N
CR = 2048
NCH = R // CR
NBUF = 8
D = 4


def _body(x_ref, o_ref, buf, in_sem, out_sem):
    def in_copy(i, s):
        return pltpu.make_async_copy(
            x_ref.at[pl.ds(i * CR, CR), :],
            buf.at[pl.ds(s * CR, CR), :],
            in_sem.at[s],
        )

    def out_copy(i, s):
        return pltpu.make_async_copy(
            buf.at[pl.ds(s * CR, CR), :],
            o_ref.at[pl.ds(i * CR, CR), :],
            out_sem.at[s],
        )

    for j in range(D):
        in_copy(j, j % NBUF).start()

    zi = lax.broadcasted_iota(jnp.int32, (CR, N), 1)
    for i in range(NCH):
        s = i % NBUF
        in_copy(i, s).wait()
        x = buf[pl.ds(s * CR, CR), :]
        buf[pl.ds(s * CR, CR), :] = jnp.where(zi == 0, x[:, 1:2], x)
        out_copy(i, s).start()
        j = i + D
        if j < NCH:
            k = j - NBUF
            if k >= 0:
                out_copy(k, k % NBUF).wait()
            in_copy(j, j % NBUF).start()
    for k in range(max(0, NCH - NBUF), NCH):
        out_copy(k, k % NBUF).wait()


def kernel(b):
    bs = b.reshape(R, N)
    out = pl.pallas_call(
        _body,
        in_specs=[pl.BlockSpec(memory_space=pltpu.MemorySpace.HBM)],
        out_specs=pl.BlockSpec(memory_space=pltpu.MemorySpace.HBM),
        out_shape=jax.ShapeDtypeStruct((R, N), jnp.float32),
        scratch_shapes=[
            pltpu.VMEM((NBUF * CR, N), jnp.float32),
            pltpu.SemaphoreType.DMA((NBUF,)),
            pltpu.SemaphoreType.DMA((NBUF,)),
        ],
    )(bs)
    return out.reshape(1, N, N, N, 1)
